# initial kernel scaffold (unmeasured)
import jax
import jax.numpy as jnp
from jax import lax
from jax.experimental import pallas as pl
from jax.experimental.pallas import tpu as pltpu

N_DEV = 4
SQ = 2048
SKV = 2048
DH = 128
H_LOC = 8
DM = 1024
BQ = 512
SCALE = 0.08838834764831843
BLK = 64


def _attn_body(x_ref, wq_ref, k_ref, v_ref, wo_ref, out_ref):
    r = pl.program_id(0)
    h = pl.program_id(1)

    q = jnp.dot(x_ref[...], wq_ref[...], preferred_element_type=jnp.float32)
    k = k_ref[:, 0, :]
    s = lax.dot_general(
        q, k,
        dimension_numbers=(((1,), (1,)), ((), ())),
        preferred_element_type=jnp.float32,
    ) * SCALE

    qb = (lax.broadcasted_iota(jnp.int32, (BQ, SKV), 0) + r * BQ) // BLK
    kb = lax.broadcasted_iota(jnp.int32, (BQ, SKV), 1) // BLK
    mask = (qb == kb) | (kb == 0) | ((qb + kb) % 3 == 0)
    s = jnp.where(mask, s, -1e9)

    m = jnp.max(s, axis=1, keepdims=True)
    w = jnp.exp(s - m)
    w = w / jnp.sum(w, axis=1, keepdims=True)

    ctx = jnp.dot(w, v_ref[:, 0, :], preferred_element_type=jnp.float32)
    p = jnp.dot(ctx, wo_ref[...], preferred_element_type=jnp.float32)

    @pl.when(h == 0)
    def _():
        out_ref[...] = p

    @pl.when(h > 0)
    def _():
        out_ref[...] = out_ref[...] + p


def _attn_partial(x2, Wq, Kh, Vh, Wo):
    return pl.pallas_call(
        _attn_body,
        grid=(SQ // BQ, H_LOC),
        in_specs=[
            pl.BlockSpec((BQ, DM), lambda r, h: (r, 0)),
            pl.BlockSpec((DM, DH), lambda r, h: (0, h)),
            pl.BlockSpec((SKV, 1, DH), lambda r, h: (0, h, 0)),
            pl.BlockSpec((SKV, 1, DH), lambda r, h: (0, h, 0)),
            pl.BlockSpec((DH, DM), lambda r, h: (h, 0)),
        ],
        out_specs=pl.BlockSpec((BQ, DM), lambda r, h: (r, 0)),
        out_shape=jax.ShapeDtypeStruct((SQ, DM), jnp.float32),
    )(x2, Wq, Kh, Vh, Wo)


def _ar_body(p_ref, out_ref, comm_ref, send_sems, recv_sems):
    my = lax.axis_index("i")
    left = lax.rem(my + N_DEV - 1, N_DEV)
    right = lax.rem(my + 1, N_DEV)

    barrier = pltpu.get_barrier_semaphore()
    for nbr in (left, right):
        pl.semaphore_signal(
            barrier, inc=1,
            device_id=(nbr,), device_id_type=pl.DeviceIdType.MESH,
        )
    pl.semaphore_wait(barrier, 2)

    out_ref[...] = p_ref[...]
    comm_ref[0, :, :] = p_ref[...]

    for h in range(N_DEV - 1):
        send_slot = h % 2
        recv_slot = (h + 1) % 2
        rdma = pltpu.make_async_remote_copy(
            src_ref=comm_ref.at[send_slot],
            dst_ref=comm_ref.at[recv_slot],
            send_sem=send_sems.at[send_slot],
            recv_sem=recv_sems.at[recv_slot],
            device_id=(right,),
            device_id_type=pl.DeviceIdType.MESH,
        )
        rdma.start()
        rdma.wait()
        out_ref[...] = out_ref[...] + comm_ref[recv_slot, :, :]


def _ring_allreduce(partial):
    return pl.pallas_call(
        _ar_body,
        out_shape=jax.ShapeDtypeStruct((SQ, DM), jnp.float32),
        in_specs=[pl.BlockSpec(memory_space=pltpu.VMEM)],
        out_specs=pl.BlockSpec(memory_space=pltpu.VMEM),
        scratch_shapes=[
            pltpu.VMEM((2, SQ, DM), jnp.float32),
            pltpu.SemaphoreType.DMA((2,)),
            pltpu.SemaphoreType.DMA((2,)),
        ],
        compiler_params=pltpu.CompilerParams(collective_id=0),
    )(partial)


def kernel(x, Wq, K_ext, V_ext, Wo):
    x2 = x.reshape(SQ, DM)
    my_i = lax.axis_index("i")
    K3 = K_ext.reshape(SKV, N_DEV * H_LOC, DH)
    V3 = V_ext.reshape(SKV, N_DEV * H_LOC, DH)
    Kh = lax.dynamic_slice_in_dim(K3, my_i * H_LOC, H_LOC, axis=1)
    Vh = lax.dynamic_slice_in_dim(V3, my_i * H_LOC, H_LOC, axis=1)

    partial = _attn_partial(x2, Wq, Kh, Vh, Wo)
    out = _ring_allreduce(partial)
    return out.reshape(1, SQ, DM)


# baseline (device time: 507816 ns/iter reference)
import jax
import jax.numpy as jnp
from jax import lax
from jax.experimental import pallas as pl
from jax.experimental.pallas import tpu as pltpu

N_DEV = 4
SQ = 2048
SKV = 2048
DH = 128
H_LOC = 8
DM = 1024
BQ = 512
SCALE = 0.08838834764831843
BLK = 64


def _attn_body(x_ref, wq_ref, k_ref, v_ref, wo_ref, out_ref):
    r = pl.program_id(0)
    h = pl.program_id(1)

    q = jnp.dot(x_ref[...], wq_ref[...], preferred_element_type=jnp.float32)
    k = k_ref[0, :, :]
    s = lax.dot_general(
        q, k,
        dimension_numbers=(((1,), (1,)), ((), ())),
        preferred_element_type=jnp.float32,
    ) * SCALE

    qb = (lax.broadcasted_iota(jnp.int32, (BQ, SKV), 0) + r * BQ) // BLK
    kb = lax.broadcasted_iota(jnp.int32, (BQ, SKV), 1) // BLK
    mask = (qb == kb) | (kb == 0) | ((qb + kb) % 3 == 0)
    s = jnp.where(mask, s, -1e9)

    m = jnp.max(s, axis=1, keepdims=True)
    w = jnp.exp(s - m)
    w = w / jnp.sum(w, axis=1, keepdims=True)

    ctx = jnp.dot(w, v_ref[0, :, :], preferred_element_type=jnp.float32)
    p = jnp.dot(ctx, wo_ref[...], preferred_element_type=jnp.float32)

    @pl.when(h == 0)
    def _():
        out_ref[...] = p

    @pl.when(h > 0)
    def _():
        out_ref[...] = out_ref[...] + p


def _attn_partial(x2, Wq, Kh, Vh, Wo):
    return pl.pallas_call(
        _attn_body,
        grid=(SQ // BQ, H_LOC),
        in_specs=[
            pl.BlockSpec((BQ, DM), lambda r, h: (r, 0)),
            pl.BlockSpec((DM, DH), lambda r, h: (0, h)),
            pl.BlockSpec((1, SKV, DH), lambda r, h: (h, 0, 0)),
            pl.BlockSpec((1, SKV, DH), lambda r, h: (h, 0, 0)),
            pl.BlockSpec((DH, DM), lambda r, h: (h, 0)),
        ],
        out_specs=pl.BlockSpec((BQ, DM), lambda r, h: (r, 0)),
        out_shape=jax.ShapeDtypeStruct((SQ, DM), jnp.float32),
    )(x2, Wq, Kh, Vh, Wo)


def _ar_body(p_ref, out_ref, comm_ref, send_sems, recv_sems):
    my = lax.axis_index("i")
    left = lax.rem(my + N_DEV - 1, N_DEV)
    right = lax.rem(my + 1, N_DEV)

    barrier = pltpu.get_barrier_semaphore()
    for nbr in (left, right):
        pl.semaphore_signal(
            barrier, inc=1,
            device_id=(nbr,), device_id_type=pl.DeviceIdType.MESH,
        )
    pl.semaphore_wait(barrier, 2)

    out_ref[...] = p_ref[...]
    comm_ref[0, :, :] = p_ref[...]

    for h in range(N_DEV - 1):
        send_slot = h % 2
        recv_slot = (h + 1) % 2
        rdma = pltpu.make_async_remote_copy(
            src_ref=comm_ref.at[send_slot],
            dst_ref=comm_ref.at[recv_slot],
            send_sem=send_sems.at[send_slot],
            recv_sem=recv_sems.at[recv_slot],
            device_id=(right,),
            device_id_type=pl.DeviceIdType.MESH,
        )
        rdma.start()
        rdma.wait()
        out_ref[...] = out_ref[...] + comm_ref[recv_slot, :, :]


def _ring_allreduce(partial):
    return pl.pallas_call(
        _ar_body,
        out_shape=jax.ShapeDtypeStruct((SQ, DM), jnp.float32),
        in_specs=[pl.BlockSpec(memory_space=pltpu.VMEM)],
        out_specs=pl.BlockSpec(memory_space=pltpu.VMEM),
        scratch_shapes=[
            pltpu.VMEM((2, SQ, DM), jnp.float32),
            pltpu.SemaphoreType.DMA((2,)),
            pltpu.SemaphoreType.DMA((2,)),
        ],
        compiler_params=pltpu.CompilerParams(collective_id=0),
    )(partial)


def kernel(x, Wq, K_ext, V_ext, Wo):
    x2 = x.reshape(SQ, DM)
    my_i = lax.axis_index("i")
    K3 = K_ext.reshape(SKV, N_DEV * H_LOC, DH)
    V3 = V_ext.reshape(SKV, N_DEV * H_LOC, DH)
    Kh = lax.dynamic_slice_in_dim(K3, my_i * H_LOC, H_LOC, axis=1).transpose(1, 0, 2)
    Vh = lax.dynamic_slice_in_dim(V3, my_i * H_LOC, H_LOC, axis=1).transpose(1, 0, 2)

    partial = _attn_partial(x2, Wq, Kh, Vh, Wo)
    out = _ring_allreduce(partial)
    return out.reshape(1, SQ, DM)


# device time: 324435 ns/iter; 1.5652x vs baseline; 1.5652x over previous
import jax
import jax.numpy as jnp
from jax import lax
from jax.experimental import pallas as pl
from jax.experimental.pallas import tpu as pltpu

N_DEV = 4
SQ = 2048
SKV = 2048
DH = 128
H_LOC = 8
DM = 1024
BQ = 512
SCALE = 0.08838834764831843
BLK = 64


def _attn_body(x_ref, wq_ref, k_ref, v_ref, wo_ref, out_ref):
    r = pl.program_id(0)
    h = pl.program_id(1)

    q = jnp.dot(x_ref[...], wq_ref[...], preferred_element_type=jnp.float32)
    k = k_ref[0, :, :]
    s = lax.dot_general(
        q, k,
        dimension_numbers=(((1,), (1,)), ((), ())),
        preferred_element_type=jnp.float32,
    ) * SCALE

    qb = (lax.broadcasted_iota(jnp.int32, (BQ, SKV), 0) + r * BQ) // BLK
    kb = lax.broadcasted_iota(jnp.int32, (BQ, SKV), 1) // BLK
    mask = (qb == kb) | (kb == 0) | ((qb + kb) % 3 == 0)
    s = jnp.where(mask, s, -1e9)

    m = jnp.max(s, axis=1, keepdims=True)
    w = jnp.exp(s - m)
    w = w / jnp.sum(w, axis=1, keepdims=True)

    ctx = jnp.dot(w, v_ref[0, :, :], preferred_element_type=jnp.float32)
    p = jnp.dot(ctx, wo_ref[...], preferred_element_type=jnp.float32)

    @pl.when(h == 0)
    def _():
        out_ref[...] = p

    @pl.when(h > 0)
    def _():
        out_ref[...] = out_ref[...] + p


def _attn_partial(x2, Wq, Kh, Vh, Wo):
    return pl.pallas_call(
        _attn_body,
        grid=(SQ // BQ, H_LOC),
        in_specs=[
            pl.BlockSpec((BQ, DM), lambda r, h: (r, 0)),
            pl.BlockSpec((DM, DH), lambda r, h: (0, h)),
            pl.BlockSpec((1, SKV, DH), lambda r, h: (h, 0, 0)),
            pl.BlockSpec((1, SKV, DH), lambda r, h: (h, 0, 0)),
            pl.BlockSpec((DH, DM), lambda r, h: (h, 0)),
        ],
        out_specs=pl.BlockSpec((BQ, DM), lambda r, h: (r, 0)),
        out_shape=jax.ShapeDtypeStruct((SQ, DM), jnp.float32),
    )(x2, Wq, Kh, Vh, Wo)


CH = SQ // N_DEV


def _ar_body(p_ref, out_ref, rs_buf, rs_send_sems, rs_recv_sems,
             ag_send_sems, ag_recv_sems):
    my = lax.axis_index("i")

    barrier = pltpu.get_barrier_semaphore()
    for d in range(1, N_DEV):
        peer = lax.rem(my + d, N_DEV)
        pl.semaphore_signal(
            barrier, inc=1,
            device_id=(peer,), device_id_type=pl.DeviceIdType.MESH,
        )
    pl.semaphore_wait(barrier, N_DEV - 1)

    rs_sends = []
    for d in range(1, N_DEV):
        peer = lax.rem(my + d, N_DEV)
        rdma = pltpu.make_async_remote_copy(
            src_ref=p_ref.at[pl.ds(peer * CH, CH)],
            dst_ref=rs_buf.at[my],
            send_sem=rs_send_sems.at[d],
            recv_sem=rs_recv_sems.at[my],
            device_id=(peer,),
            device_id_type=pl.DeviceIdType.MESH,
        )
        rdma.start()
        rs_sends.append(rdma)

    acc = p_ref[pl.ds(my * CH, CH), :]
    for d in range(1, N_DEV):
        src = lax.rem(my + d, N_DEV)
        recv = pltpu.make_async_remote_copy(
            src_ref=rs_buf.at[src],
            dst_ref=rs_buf.at[src],
            send_sem=rs_send_sems.at[d],
            recv_sem=rs_recv_sems.at[src],
            device_id=(src,),
            device_id_type=pl.DeviceIdType.MESH,
        )
        recv.wait_recv()
        acc = acc + rs_buf[src, :, :]
    out_ref[pl.ds(my * CH, CH), :] = acc

    for rdma in rs_sends:
        rdma.wait_send()

    ag_sends = []
    for d in range(1, N_DEV):
        peer = lax.rem(my + d, N_DEV)
        rdma = pltpu.make_async_remote_copy(
            src_ref=out_ref.at[pl.ds(my * CH, CH)],
            dst_ref=out_ref.at[pl.ds(my * CH, CH)],
            send_sem=ag_send_sems.at[d],
            recv_sem=ag_recv_sems.at[my],
            device_id=(peer,),
            device_id_type=pl.DeviceIdType.MESH,
        )
        rdma.start()
        ag_sends.append(rdma)

    for d in range(1, N_DEV):
        src = lax.rem(my + d, N_DEV)
        recv = pltpu.make_async_remote_copy(
            src_ref=out_ref.at[pl.ds(src * CH, CH)],
            dst_ref=out_ref.at[pl.ds(src * CH, CH)],
            send_sem=ag_send_sems.at[d],
            recv_sem=ag_recv_sems.at[src],
            device_id=(src,),
            device_id_type=pl.DeviceIdType.MESH,
        )
        recv.wait_recv()

    for rdma in ag_sends:
        rdma.wait_send()


def _ring_allreduce(partial):
    return pl.pallas_call(
        _ar_body,
        out_shape=jax.ShapeDtypeStruct((SQ, DM), jnp.float32),
        in_specs=[pl.BlockSpec(memory_space=pltpu.VMEM)],
        out_specs=pl.BlockSpec(memory_space=pltpu.VMEM),
        scratch_shapes=[
            pltpu.VMEM((N_DEV, CH, DM), jnp.float32),
            pltpu.SemaphoreType.DMA((N_DEV,)),
            pltpu.SemaphoreType.DMA((N_DEV,)),
            pltpu.SemaphoreType.DMA((N_DEV,)),
            pltpu.SemaphoreType.DMA((N_DEV,)),
        ],
        compiler_params=pltpu.CompilerParams(collective_id=0),
    )(partial)


def kernel(x, Wq, K_ext, V_ext, Wo):
    x2 = x.reshape(SQ, DM)
    my_i = lax.axis_index("i")
    K3 = K_ext.reshape(SKV, N_DEV * H_LOC, DH)
    V3 = V_ext.reshape(SKV, N_DEV * H_LOC, DH)
    Kh = lax.dynamic_slice_in_dim(K3, my_i * H_LOC, H_LOC, axis=1).transpose(1, 0, 2)
    Vh = lax.dynamic_slice_in_dim(V3, my_i * H_LOC, H_LOC, axis=1).transpose(1, 0, 2)

    partial = _attn_partial(x2, Wq, Kh, Vh, Wo)
    out = _ring_allreduce(partial)
    return out.reshape(1, SQ, DM)


# device time: 288534 ns/iter; 1.7600x vs baseline; 1.1244x over previous
import numpy as np

import jax
import jax.numpy as jnp
from jax import lax
from jax.experimental import pallas as pl
from jax.experimental.pallas import tpu as pltpu

N_DEV = 4
SQ = 2048
SKV = 2048
DH = 128
H_LOC = 8
DM = 1024
BQ = 512
SCALE = 0.08838834764831843
BLK = 64


NKB = 13


def _build_tables():
    nb = SKV // BLK
    tbl = np.zeros((nb, NKB), np.int32)
    nv = np.zeros((nb,), np.int32)
    for qb in range(nb):
        allowed = sorted({qb, 0} | {kb for kb in range(nb) if (qb + kb) % 3 == 0})
        nv[qb] = len(allowed)
        tbl[qb] = allowed + [allowed[-1]] * (NKB - len(allowed))
    return jnp.asarray(tbl), jnp.asarray(nv)


def _attn_body(tbl_ref, nv_ref, x_ref, wq_ref, k_ref, v_ref, wo_ref, out_ref):
    r = pl.program_id(0)
    h = pl.program_id(1)

    q_all = jnp.dot(x_ref[...], wq_ref[...], preferred_element_type=jnp.float32)

    ctxs = []
    for j in range(BQ // BLK):
        qb = r * (BQ // BLK) + j
        kp = jnp.concatenate(
            [k_ref[0, pl.ds(tbl_ref[qb, t] * BLK, BLK), :] for t in range(NKB)],
            axis=0,
        )
        q = q_all[j * BLK:(j + 1) * BLK, :]
        s = lax.dot_general(
            q, kp,
            dimension_numbers=(((1,), (1,)), ((), ())),
            preferred_element_type=jnp.float32,
        ) * SCALE
        col = lax.broadcasted_iota(jnp.int32, (BLK, NKB * BLK), 1)
        s = jnp.where(col < nv_ref[qb] * BLK, s, -1e9)
        m = jnp.max(s, axis=1, keepdims=True)
        w = jnp.exp(s - m)
        w = w / jnp.sum(w, axis=1, keepdims=True)
        vp = jnp.concatenate(
            [v_ref[0, pl.ds(tbl_ref[qb, t] * BLK, BLK), :] for t in range(NKB)],
            axis=0,
        )
        ctxs.append(jnp.dot(w, vp, preferred_element_type=jnp.float32))

    ctx = jnp.concatenate(ctxs, axis=0)
    p = jnp.dot(ctx, wo_ref[...], preferred_element_type=jnp.float32)

    @pl.when(h == 0)
    def _():
        out_ref[...] = p

    @pl.when(h > 0)
    def _():
        out_ref[...] = out_ref[...] + p


def _attn_partial(x2, Wq, Kh, Vh, Wo):
    tbl, nv = _build_tables()
    grid_spec = pltpu.PrefetchScalarGridSpec(
        num_scalar_prefetch=2,
        grid=(SQ // BQ, H_LOC),
        in_specs=[
            pl.BlockSpec((BQ, DM), lambda r, h, *_: (r, 0)),
            pl.BlockSpec((DM, DH), lambda r, h, *_: (0, h)),
            pl.BlockSpec((1, SKV, DH), lambda r, h, *_: (h, 0, 0)),
            pl.BlockSpec((1, SKV, DH), lambda r, h, *_: (h, 0, 0)),
            pl.BlockSpec((DH, DM), lambda r, h, *_: (h, 0)),
        ],
        out_specs=pl.BlockSpec((BQ, DM), lambda r, h, *_: (r, 0)),
    )
    return pl.pallas_call(
        _attn_body,
        grid_spec=grid_spec,
        out_shape=jax.ShapeDtypeStruct((SQ, DM), jnp.float32),
    )(tbl, nv, x2, Wq, Kh, Vh, Wo)


CH = SQ // N_DEV


def _ar_body(p_ref, out_ref, rs_buf, rs_send_sems, rs_recv_sems,
             ag_send_sems, ag_recv_sems):
    my = lax.axis_index("i")

    barrier = pltpu.get_barrier_semaphore()
    for d in range(1, N_DEV):
        peer = lax.rem(my + d, N_DEV)
        pl.semaphore_signal(
            barrier, inc=1,
            device_id=(peer,), device_id_type=pl.DeviceIdType.MESH,
        )
    pl.semaphore_wait(barrier, N_DEV - 1)

    rs_sends = []
    for d in range(1, N_DEV):
        peer = lax.rem(my + d, N_DEV)
        rdma = pltpu.make_async_remote_copy(
            src_ref=p_ref.at[pl.ds(peer * CH, CH)],
            dst_ref=rs_buf.at[my],
            send_sem=rs_send_sems.at[d],
            recv_sem=rs_recv_sems.at[my],
            device_id=(peer,),
            device_id_type=pl.DeviceIdType.MESH,
        )
        rdma.start()
        rs_sends.append(rdma)

    acc = p_ref[pl.ds(my * CH, CH), :]
    for d in range(1, N_DEV):
        src = lax.rem(my + d, N_DEV)
        recv = pltpu.make_async_remote_copy(
            src_ref=rs_buf.at[src],
            dst_ref=rs_buf.at[src],
            send_sem=rs_send_sems.at[d],
            recv_sem=rs_recv_sems.at[src],
            device_id=(src,),
            device_id_type=pl.DeviceIdType.MESH,
        )
        recv.wait_recv()
        acc = acc + rs_buf[src, :, :]
    out_ref[pl.ds(my * CH, CH), :] = acc

    for rdma in rs_sends:
        rdma.wait_send()

    ag_sends = []
    for d in range(1, N_DEV):
        peer = lax.rem(my + d, N_DEV)
        rdma = pltpu.make_async_remote_copy(
            src_ref=out_ref.at[pl.ds(my * CH, CH)],
            dst_ref=out_ref.at[pl.ds(my * CH, CH)],
            send_sem=ag_send_sems.at[d],
            recv_sem=ag_recv_sems.at[my],
            device_id=(peer,),
            device_id_type=pl.DeviceIdType.MESH,
        )
        rdma.start()
        ag_sends.append(rdma)

    for d in range(1, N_DEV):
        src = lax.rem(my + d, N_DEV)
        recv = pltpu.make_async_remote_copy(
            src_ref=out_ref.at[pl.ds(src * CH, CH)],
            dst_ref=out_ref.at[pl.ds(src * CH, CH)],
            send_sem=ag_send_sems.at[d],
            recv_sem=ag_recv_sems.at[src],
            device_id=(src,),
            device_id_type=pl.DeviceIdType.MESH,
        )
        recv.wait_recv()

    for rdma in ag_sends:
        rdma.wait_send()


def _ring_allreduce(partial):
    return pl.pallas_call(
        _ar_body,
        out_shape=jax.ShapeDtypeStruct((SQ, DM), jnp.float32),
        in_specs=[pl.BlockSpec(memory_space=pltpu.VMEM)],
        out_specs=pl.BlockSpec(memory_space=pltpu.VMEM),
        scratch_shapes=[
            pltpu.VMEM((N_DEV, CH, DM), jnp.float32),
            pltpu.SemaphoreType.DMA((N_DEV,)),
            pltpu.SemaphoreType.DMA((N_DEV,)),
            pltpu.SemaphoreType.DMA((N_DEV,)),
            pltpu.SemaphoreType.DMA((N_DEV,)),
        ],
        compiler_params=pltpu.CompilerParams(collective_id=0),
    )(partial)


def kernel(x, Wq, K_ext, V_ext, Wo):
    x2 = x.reshape(SQ, DM)
    my_i = lax.axis_index("i")
    K3 = K_ext.reshape(SKV, N_DEV * H_LOC, DH)
    V3 = V_ext.reshape(SKV, N_DEV * H_LOC, DH)
    Kh = lax.dynamic_slice_in_dim(K3, my_i * H_LOC, H_LOC, axis=1).transpose(1, 0, 2)
    Vh = lax.dynamic_slice_in_dim(V3, my_i * H_LOC, H_LOC, axis=1).transpose(1, 0, 2)

    partial = _attn_partial(x2, Wq, Kh, Vh, Wo)
    out = _ring_allreduce(partial)
    return out.reshape(1, SQ, DM)


# device time: 287225 ns/iter; 1.7680x vs baseline; 1.0046x over previous
import numpy as np

import jax
import jax.numpy as jnp
from jax import lax
from jax.experimental import pallas as pl
from jax.experimental.pallas import tpu as pltpu

N_DEV = 4
SQ = 2048
SKV = 2048
DH = 128
H_LOC = 8
DM = 1024
BQ = 512
SCALE = 0.08838834764831843
BLK = 64


NKB = 13


def _build_tables():
    nb = SKV // BLK
    tbl = np.zeros((nb, NKB), np.int32)
    nv = np.zeros((nb,), np.int32)
    for qb in range(nb):
        allowed = sorted({qb, 0} | {kb for kb in range(nb) if (qb + kb) % 3 == 0})
        nv[qb] = len(allowed)
        tbl[qb] = allowed + [allowed[-1]] * (NKB - len(allowed))
    return jnp.asarray(tbl), jnp.asarray(nv)


def _attn_body(tbl_ref, nv_ref, x_ref, wq_ref, k_ref, v_ref, wo_ref, out_ref):
    r = pl.program_id(0)
    h = pl.program_id(1)

    q_all = jnp.dot(
        x_ref[...], wq_ref[...], preferred_element_type=jnp.float32
    ).astype(jnp.bfloat16)

    ctxs = []
    for j in range(BQ // BLK):
        qb = r * (BQ // BLK) + j
        kp = jnp.concatenate(
            [k_ref[0, pl.ds(tbl_ref[qb, t] * BLK, BLK), :] for t in range(NKB)],
            axis=0,
        )
        q = q_all[j * BLK:(j + 1) * BLK, :]
        s = lax.dot_general(
            q, kp,
            dimension_numbers=(((1,), (1,)), ((), ())),
            preferred_element_type=jnp.float32,
        ) * SCALE
        col = lax.broadcasted_iota(jnp.int32, (BLK, NKB * BLK), 1)
        s = jnp.where(col < nv_ref[qb] * BLK, s, -1e9)
        m = jnp.max(s, axis=1, keepdims=True)
        w = jnp.exp(s - m)
        w = (w / jnp.sum(w, axis=1, keepdims=True)).astype(jnp.bfloat16)
        vp = jnp.concatenate(
            [v_ref[0, pl.ds(tbl_ref[qb, t] * BLK, BLK), :] for t in range(NKB)],
            axis=0,
        )
        ctxs.append(jnp.dot(w, vp, preferred_element_type=jnp.float32))

    ctx = jnp.concatenate(ctxs, axis=0).astype(jnp.bfloat16)
    p = jnp.dot(ctx, wo_ref[...], preferred_element_type=jnp.float32)

    @pl.when(h == 0)
    def _():
        out_ref[...] = p

    @pl.when(h > 0)
    def _():
        out_ref[...] = out_ref[...] + p


def _attn_partial(x2, Wq, Kh, Vh, Wo):
    tbl, nv = _build_tables()
    grid_spec = pltpu.PrefetchScalarGridSpec(
        num_scalar_prefetch=2,
        grid=(SQ // BQ, H_LOC),
        in_specs=[
            pl.BlockSpec((BQ, DM), lambda r, h, *_: (r, 0)),
            pl.BlockSpec((DM, DH), lambda r, h, *_: (0, h)),
            pl.BlockSpec((1, SKV, DH), lambda r, h, *_: (h, 0, 0)),
            pl.BlockSpec((1, SKV, DH), lambda r, h, *_: (h, 0, 0)),
            pl.BlockSpec((DH, DM), lambda r, h, *_: (h, 0)),
        ],
        out_specs=pl.BlockSpec((BQ, DM), lambda r, h, *_: (r, 0)),
    )
    return pl.pallas_call(
        _attn_body,
        grid_spec=grid_spec,
        out_shape=jax.ShapeDtypeStruct((SQ, DM), jnp.float32),
    )(tbl, nv, x2, Wq, Kh, Vh, Wo)


CH = SQ // N_DEV


def _ar_body(p_ref, out_ref, rs_buf, rs_send_sems, rs_recv_sems,
             ag_send_sems, ag_recv_sems):
    my = lax.axis_index("i")

    barrier = pltpu.get_barrier_semaphore()
    for d in range(1, N_DEV):
        peer = lax.rem(my + d, N_DEV)
        pl.semaphore_signal(
            barrier, inc=1,
            device_id=(peer,), device_id_type=pl.DeviceIdType.MESH,
        )
    pl.semaphore_wait(barrier, N_DEV - 1)

    rs_sends = []
    for d in range(1, N_DEV):
        peer = lax.rem(my + d, N_DEV)
        rdma = pltpu.make_async_remote_copy(
            src_ref=p_ref.at[pl.ds(peer * CH, CH)],
            dst_ref=rs_buf.at[my],
            send_sem=rs_send_sems.at[d],
            recv_sem=rs_recv_sems.at[my],
            device_id=(peer,),
            device_id_type=pl.DeviceIdType.MESH,
        )
        rdma.start()
        rs_sends.append(rdma)

    acc = p_ref[pl.ds(my * CH, CH), :]
    for d in range(1, N_DEV):
        src = lax.rem(my + d, N_DEV)
        recv = pltpu.make_async_remote_copy(
            src_ref=rs_buf.at[src],
            dst_ref=rs_buf.at[src],
            send_sem=rs_send_sems.at[d],
            recv_sem=rs_recv_sems.at[src],
            device_id=(src,),
            device_id_type=pl.DeviceIdType.MESH,
        )
        recv.wait_recv()
        acc = acc + rs_buf[src, :, :]
    out_ref[pl.ds(my * CH, CH), :] = acc

    for rdma in rs_sends:
        rdma.wait_send()

    ag_sends = []
    for d in range(1, N_DEV):
        peer = lax.rem(my + d, N_DEV)
        rdma = pltpu.make_async_remote_copy(
            src_ref=out_ref.at[pl.ds(my * CH, CH)],
            dst_ref=out_ref.at[pl.ds(my * CH, CH)],
            send_sem=ag_send_sems.at[d],
            recv_sem=ag_recv_sems.at[my],
            device_id=(peer,),
            device_id_type=pl.DeviceIdType.MESH,
        )
        rdma.start()
        ag_sends.append(rdma)

    for d in range(1, N_DEV):
        src = lax.rem(my + d, N_DEV)
        recv = pltpu.make_async_remote_copy(
            src_ref=out_ref.at[pl.ds(src * CH, CH)],
            dst_ref=out_ref.at[pl.ds(src * CH, CH)],
            send_sem=ag_send_sems.at[d],
            recv_sem=ag_recv_sems.at[src],
            device_id=(src,),
            device_id_type=pl.DeviceIdType.MESH,
        )
        recv.wait_recv()

    for rdma in ag_sends:
        rdma.wait_send()


def _ring_allreduce(partial):
    return pl.pallas_call(
        _ar_body,
        out_shape=jax.ShapeDtypeStruct((SQ, DM), jnp.float32),
        in_specs=[pl.BlockSpec(memory_space=pltpu.VMEM)],
        out_specs=pl.BlockSpec(memory_space=pltpu.VMEM),
        scratch_shapes=[
            pltpu.VMEM((N_DEV, CH, DM), jnp.float32),
            pltpu.SemaphoreType.DMA((N_DEV,)),
            pltpu.SemaphoreType.DMA((N_DEV,)),
            pltpu.SemaphoreType.DMA((N_DEV,)),
            pltpu.SemaphoreType.DMA((N_DEV,)),
        ],
        compiler_params=pltpu.CompilerParams(collective_id=0),
    )(partial)


def kernel(x, Wq, K_ext, V_ext, Wo):
    x2 = x.reshape(SQ, DM).astype(jnp.bfloat16)
    Wq = Wq.astype(jnp.bfloat16)
    Wo = Wo.astype(jnp.bfloat16)
    my_i = lax.axis_index("i")
    K3 = K_ext.reshape(SKV, N_DEV * H_LOC, DH).astype(jnp.bfloat16)
    V3 = V_ext.reshape(SKV, N_DEV * H_LOC, DH).astype(jnp.bfloat16)
    Kh = lax.dynamic_slice_in_dim(K3, my_i * H_LOC, H_LOC, axis=1).transpose(1, 0, 2)
    Vh = lax.dynamic_slice_in_dim(V3, my_i * H_LOC, H_LOC, axis=1).transpose(1, 0, 2)

    partial = _attn_partial(x2, Wq, Kh, Vh, Wo)
    out = _ring_allreduce(partial)
    return out.reshape(1, SQ, DM)


# device time: 104055 ns/iter; 4.8803x vs baseline; 2.7603x over previous
import numpy as np

import jax
import jax.numpy as jnp
from jax import lax
from jax.experimental import pallas as pl
from jax.experimental.pallas import tpu as pltpu

N_DEV = 4
SQ = 2048
SKV = 2048
DH = 128
H_LOC = 8
DM = 1024
BQ = 512
SCALE = 0.08838834764831843
BLK = 64


NKB = 13


def _build_tables():
    nb = SKV // BLK
    tbl = np.zeros((nb, NKB), np.int32)
    nv = np.zeros((nb,), np.int32)
    for qb in range(nb):
        allowed = sorted({qb, 0} | {kb for kb in range(nb) if (qb + kb) % 3 == 0})
        nv[qb] = len(allowed)
        tbl[qb] = allowed + [allowed[-1]] * (NKB - len(allowed))
    return jnp.asarray(tbl), jnp.asarray(nv)


def _attn_body(tbl_ref, nv_ref, x_ref, wq_ref, k_ref, v_ref, wo_ref, out_ref):
    r = pl.program_id(0)
    h = pl.program_id(1)

    q_all = jnp.dot(
        x_ref[...], wq_ref[...], preferred_element_type=jnp.float32
    ).astype(jnp.bfloat16)

    ctxs = []
    for j in range(BQ // BLK):
        qb = r * (BQ // BLK) + j
        kp = jnp.concatenate(
            [k_ref[0, pl.ds(tbl_ref[qb, t] * BLK, BLK), :] for t in range(NKB)],
            axis=0,
        )
        q = q_all[j * BLK:(j + 1) * BLK, :]
        s = lax.dot_general(
            q, kp,
            dimension_numbers=(((1,), (1,)), ((), ())),
            preferred_element_type=jnp.float32,
        ) * SCALE
        col = lax.broadcasted_iota(jnp.int32, (BLK, NKB * BLK), 1)
        s = jnp.where(col < nv_ref[qb] * BLK, s, -1e9)
        m = jnp.max(s, axis=1, keepdims=True)
        w = jnp.exp(s - m)
        w = (w / jnp.sum(w, axis=1, keepdims=True)).astype(jnp.bfloat16)
        vp = jnp.concatenate(
            [v_ref[0, pl.ds(tbl_ref[qb, t] * BLK, BLK), :] for t in range(NKB)],
            axis=0,
        )
        ctxs.append(jnp.dot(w, vp, preferred_element_type=jnp.float32))

    ctx = jnp.concatenate(ctxs, axis=0).astype(jnp.bfloat16)
    p = jnp.dot(ctx, wo_ref[...], preferred_element_type=jnp.float32)

    @pl.when(h == 0)
    def _():
        out_ref[...] = p

    @pl.when(h > 0)
    def _():
        out_ref[...] = out_ref[...] + p


def _attn_partial(x2, Wq, Kh, Vh, Wo):
    tbl, nv = _build_tables()
    grid_spec = pltpu.PrefetchScalarGridSpec(
        num_scalar_prefetch=2,
        grid=(SQ // BQ, H_LOC),
        in_specs=[
            pl.BlockSpec((BQ, DM), lambda r, h, *_: (r, 0)),
            pl.BlockSpec((DM, DH), lambda r, h, *_: (0, h)),
            pl.BlockSpec((1, SKV, DH), lambda r, h, *_: (h, 0, 0)),
            pl.BlockSpec((1, SKV, DH), lambda r, h, *_: (h, 0, 0)),
            pl.BlockSpec((DH, DM), lambda r, h, *_: (h, 0)),
        ],
        out_specs=pl.BlockSpec((BQ, DM), lambda r, h, *_: (r, 0)),
    )
    return pl.pallas_call(
        _attn_body,
        grid_spec=grid_spec,
        out_shape=jax.ShapeDtypeStruct((SQ, DM), jnp.float32),
    )(tbl, nv, x2, Wq, Kh, Vh, Wo)


CH = SQ // N_DEV


def _ar_body(p_ref, out_ref, rs_buf, rs_send_sems, rs_recv_sems,
             ag_send_sems, ag_recv_sems):
    my = lax.axis_index("i")

    barrier = pltpu.get_barrier_semaphore()
    for d in range(1, N_DEV):
        peer = lax.rem(my + d, N_DEV)
        pl.semaphore_signal(
            barrier, inc=1,
            device_id=(peer,), device_id_type=pl.DeviceIdType.MESH,
        )
    pl.semaphore_wait(barrier, N_DEV - 1)

    rs_sends = []
    for d in range(1, N_DEV):
        peer = lax.rem(my + d, N_DEV)
        rdma = pltpu.make_async_remote_copy(
            src_ref=p_ref.at[pl.ds(peer * CH, CH)],
            dst_ref=rs_buf.at[my],
            send_sem=rs_send_sems.at[d],
            recv_sem=rs_recv_sems.at[my],
            device_id=(peer,),
            device_id_type=pl.DeviceIdType.MESH,
        )
        rdma.start()
        rs_sends.append(rdma)

    acc = p_ref[pl.ds(my * CH, CH), :]
    for d in range(1, N_DEV):
        src = lax.rem(my + d, N_DEV)
        recv = pltpu.make_async_remote_copy(
            src_ref=rs_buf.at[src],
            dst_ref=rs_buf.at[src],
            send_sem=rs_send_sems.at[d],
            recv_sem=rs_recv_sems.at[src],
            device_id=(src,),
            device_id_type=pl.DeviceIdType.MESH,
        )
        recv.wait_recv()
        acc = acc + rs_buf[src, :, :]
    out_ref[pl.ds(my * CH, CH), :] = acc

    for rdma in rs_sends:
        rdma.wait_send()

    ag_sends = []
    for d in range(1, N_DEV):
        peer = lax.rem(my + d, N_DEV)
        rdma = pltpu.make_async_remote_copy(
            src_ref=out_ref.at[pl.ds(my * CH, CH)],
            dst_ref=out_ref.at[pl.ds(my * CH, CH)],
            send_sem=ag_send_sems.at[d],
            recv_sem=ag_recv_sems.at[my],
            device_id=(peer,),
            device_id_type=pl.DeviceIdType.MESH,
        )
        rdma.start()
        ag_sends.append(rdma)

    for d in range(1, N_DEV):
        src = lax.rem(my + d, N_DEV)
        recv = pltpu.make_async_remote_copy(
            src_ref=out_ref.at[pl.ds(src * CH, CH)],
            dst_ref=out_ref.at[pl.ds(src * CH, CH)],
            send_sem=ag_send_sems.at[d],
            recv_sem=ag_recv_sems.at[src],
            device_id=(src,),
            device_id_type=pl.DeviceIdType.MESH,
        )
        recv.wait_recv()

    for rdma in ag_sends:
        rdma.wait_send()


def _ring_allreduce(partial):
    return pl.pallas_call(
        _ar_body,
        out_shape=jax.ShapeDtypeStruct((SQ, DM), jnp.float32),
        in_specs=[pl.BlockSpec(memory_space=pltpu.VMEM)],
        out_specs=pl.BlockSpec(memory_space=pltpu.VMEM),
        scratch_shapes=[
            pltpu.VMEM((N_DEV, CH, DM), jnp.float32),
            pltpu.SemaphoreType.DMA((N_DEV,)),
            pltpu.SemaphoreType.DMA((N_DEV,)),
            pltpu.SemaphoreType.DMA((N_DEV,)),
            pltpu.SemaphoreType.DMA((N_DEV,)),
        ],
        compiler_params=pltpu.CompilerParams(collective_id=0),
    )(partial)


def kernel(x, Wq, K_ext, V_ext, Wo):
    x2 = x.reshape(SQ, DM).astype(jnp.bfloat16)
    Wq = Wq.astype(jnp.bfloat16)
    Wo = Wo.astype(jnp.bfloat16)
    my_i = lax.axis_index("i")
    K3 = K_ext.reshape(SKV, N_DEV * H_LOC, DH).astype(jnp.bfloat16)
    V3 = V_ext.reshape(SKV, N_DEV * H_LOC, DH).astype(jnp.bfloat16)
    Kh = lax.dynamic_slice_in_dim(K3, my_i * H_LOC, H_LOC, axis=1).transpose(1, 0, 2)
    Vh = lax.dynamic_slice_in_dim(V3, my_i * H_LOC, H_LOC, axis=1).transpose(1, 0, 2)

    partial = _attn_partial(x2, Wq, Kh, Vh, Wo)
    import os
    if os.environ.get("SKIP_AR"):
        return partial.reshape(1, SQ, DM)
    if os.environ.get("SKIP_ATTN"):
        out = _ring_allreduce(x.reshape(SQ, DM))
        return out.reshape(1, SQ, DM)
    out = _ring_allreduce(partial)
    return out.reshape(1, SQ, DM)
